# baseline (speedup 1.0000x reference)
import functools

import jax
import jax.numpy as jnp
from jax import lax
from jax.experimental import pallas as pl
from jax.experimental.pallas import tpu as pltpu
from jax.experimental.pallas import tpu_sc as plsc

_NEGATIVE_RATIO = 3.0
_EPS = 1e-06
_LN2 = 0.6931471805599453

_N_TOTAL = 8 * 512 * 512
_NC = 2
_NS = 16
_NW = _NC * _NS
_N_SUB = _N_TOTAL // _NW
_CHUNK = 8192
_NCHUNK = _N_SUB // _CHUNK
_UNROLL = 4
_LANES = 16

_mesh = plsc.VectorSubcoreMesh(core_axis_name="c", subcore_axis_name="s")


def _neg_log_clip(x):
    bits = plsc.bitcast(x, jnp.int32)
    e = (bits >> 23) - 127
    m = plsc.bitcast((bits & 0x007FFFFF) | 0x3F800000, jnp.float32)
    s = (m - 1.0) / (m + 1.0)
    s2 = s * s
    poly = 1.0 + s2 * ((1.0 / 3.0) + s2 * ((1.0 / 5.0)
                       + s2 * ((1.0 / 7.0) + s2 * (1.0 / 9.0))))
    lg = (2.0 * s) * poly + e.astype(jnp.float32) * _LN2
    lg = jnp.where(x > 0.0, lg, -100.0)
    return jnp.minimum(-lg, 100.0)


def _loss_parts(pbuf, gbuf, mbuf, off):
    p = pbuf[pl.ds(off, _LANES)]
    g = gbuf[pl.ds(off, _LANES)]
    m = mbuf[pl.ds(off, _LANES)]
    pos = g * m
    x = jnp.where(g > 0.5, p, 1.0 - p)
    loss = _neg_log_clip(x)
    return pos, m, loss


def _stream_loop(pred_hbm, gt_hbm, mask_hbm, bufs, sems, base, init, accum_fn):
    pbuf, gbuf, mbuf = bufs

    def fire(ci, slot):
        off = base + ci * _CHUNK
        dst = pl.ds(slot * _CHUNK, _CHUNK)
        sem = sems[slot]
        return (
            pltpu.async_copy(pred_hbm.at[pl.ds(off, _CHUNK)], pbuf.at[dst], sem),
            pltpu.async_copy(gt_hbm.at[pl.ds(off, _CHUNK)], gbuf.at[dst], sem),
            pltpu.async_copy(mask_hbm.at[pl.ds(off, _CHUNK)], mbuf.at[dst], sem),
        )

    carry = tuple(init for _ in range(_UNROLL))
    descs = fire(0, 0)
    for ci in range(_NCHUNK):
        slot = ci % 2
        for d in descs:
            d.wait()
        if ci + 1 < _NCHUNK:
            descs = fire(ci + 1, (ci + 1) % 2)

        def body(i, acc, _slot=slot):
            out = []
            for u in range(_UNROLL):
                off = _slot * _CHUNK + i + u * _LANES
                pos, m, loss = _loss_parts(pbuf, gbuf, mbuf, off)
                out.append(accum_fn(acc[u], pos, m, loss))
            return tuple(out)

        carry = plsc.parallel_loop(
            0, _CHUNK, step=_UNROLL * _LANES, carry=carry)(body)

    def _merge(*accs):
        tot = accs[0]
        for a in accs[1:]:
            tot = jax.tree.map(lambda x, y: x + y, tot, a)
        return tot

    return _merge(*carry)


@functools.partial(
    pl.kernel,
    mesh=_mesh,
    compiler_params=pltpu.CompilerParams(needs_layout_passes=False),
    out_type=jax.ShapeDtypeStruct((_NW, 4 * _LANES), jnp.float32),
    scratch_types=[
        pltpu.VMEM((2 * _CHUNK,), jnp.float32),
        pltpu.VMEM((2 * _CHUNK,), jnp.float32),
        pltpu.VMEM((2 * _CHUNK,), jnp.float32),
        pltpu.VMEM((4 * _LANES,), jnp.float32),
        pltpu.SemaphoreType.DMA,
        pltpu.SemaphoreType.DMA,
    ],
)
def _sums_kernel(pred_hbm, gt_hbm, mask_hbm, out_hbm,
                 pbuf, gbuf, mbuf, obuf, sem_a, sem_b):
    wid = lax.axis_index("s") * _NC + lax.axis_index("c")
    base = wid * _N_SUB
    z = jnp.zeros((_LANES,), jnp.float32)

    def accum(acc, pos, m, loss):
        pc, mc, pls, lsm = acc
        return (pc + pos, mc + m, pls + loss * pos, lsm + loss * m)

    pc, mc, pls, lsm = _stream_loop(
        pred_hbm, gt_hbm, mask_hbm, (pbuf, gbuf, mbuf), (sem_a, sem_b),
        base, (z, z, z, z), accum)

    obuf[pl.ds(0, _LANES)] = pc
    obuf[pl.ds(_LANES, _LANES)] = mc
    obuf[pl.ds(2 * _LANES, _LANES)] = pls
    obuf[pl.ds(3 * _LANES, _LANES)] = lsm
    pltpu.sync_copy(obuf, out_hbm.at[wid])


@functools.partial(
    pl.kernel,
    mesh=_mesh,
    compiler_params=pltpu.CompilerParams(needs_layout_passes=False),
    out_type=jax.ShapeDtypeStruct((_NW, 2 * _LANES), jnp.float32),
    scratch_types=[
        pltpu.VMEM((2 * _CHUNK,), jnp.float32),
        pltpu.VMEM((2 * _CHUNK,), jnp.float32),
        pltpu.VMEM((2 * _CHUNK,), jnp.float32),
        pltpu.VMEM((2 * _LANES,), jnp.float32),
        pltpu.VMEM((_LANES,), jnp.float32),
        pltpu.SemaphoreType.DMA,
        pltpu.SemaphoreType.DMA,
    ],
)
def _thresh_kernel(pred_hbm, gt_hbm, mask_hbm, thr_hbm, out_hbm,
                   pbuf, gbuf, mbuf, obuf, tbuf, sem_a, sem_b):
    wid = lax.axis_index("s") * _NC + lax.axis_index("c")
    base = wid * _N_SUB
    pltpu.sync_copy(thr_hbm, tbuf)
    thr = tbuf[...]
    z = jnp.zeros((_LANES,), jnp.float32)

    def accum(acc, pos, m, loss):
        cnt, ssum = acc
        v = loss * (m - pos)
        sel = v > thr
        return (cnt + jnp.where(sel, 1.0, 0.0), ssum + jnp.where(sel, v, 0.0))

    cnt, ssum = _stream_loop(
        pred_hbm, gt_hbm, mask_hbm, (pbuf, gbuf, mbuf), (sem_a, sem_b),
        base, (z, z), accum)

    obuf[pl.ds(0, _LANES)] = cnt
    obuf[pl.ds(_LANES, _LANES)] = ssum
    pltpu.sync_copy(obuf, out_hbm.at[wid])


def kernel(pred, gt, mask):
    p = pred.reshape(-1)
    g = gt.reshape(-1)
    m = mask.reshape(-1)

    parts = _sums_kernel(p, g, m).reshape(_NW, 4, _LANES)
    pos_cnt = parts[:, 0].sum()
    neg_cnt = parts[:, 1].sum() - pos_cnt
    pos_loss_sum = parts[:, 2].sum()
    neg_loss_sum = parts[:, 3].sum() - pos_loss_sum

    pos_count = jnp.floor(pos_cnt)
    neg_floor = jnp.floor(neg_cnt)
    negative_count = jnp.minimum(neg_floor, jnp.floor(pos_count * _NEGATIVE_RATIO))

    def scan_at(bits_i32):
        t = lax.bitcast_convert_type(bits_i32, jnp.float32)
        sp = _thresh_kernel(p, g, m, jnp.broadcast_to(t, (_LANES,)))
        sp = sp.reshape(_NW, 2, _LANES)
        return sp[:, 0].sum(), sp[:, 1].sum()

    def topk_rare():
        def bs(_, lohi):
            lo, hi = lohi
            mid = (lo + hi) // 2
            cnt, _s = scan_at(mid)
            take = cnt < negative_count
            return (jnp.where(take, lo, mid + 1), jnp.where(take, mid, hi))

        lo, _ = lax.fori_loop(0, 31, bs, (jnp.int32(0), jnp.int32(0x7F800000)))
        vk = lax.bitcast_convert_type(lo, jnp.float32)
        cnt, ssum = scan_at(lo)
        val = ssum + (negative_count - cnt) * vk
        return jnp.where(negative_count > 0.0, val, 0.0)

    def topk_fast():
        return neg_loss_sum

    neg_top_sum = lax.cond(negative_count < neg_floor, topk_rare, topk_fast)
    return (pos_loss_sum + neg_top_sum) / (pos_count + negative_count + _EPS)


# speedup vs baseline: 1.6761x; 1.6761x over previous
import functools

import jax
import jax.numpy as jnp
from jax import lax
from jax.experimental import pallas as pl
from jax.experimental.pallas import tpu as pltpu
from jax.experimental.pallas import tpu_sc as plsc

_NEGATIVE_RATIO = 3.0
_EPS = 1e-06
_LN2 = 0.6931471805599453

_N_TOTAL = 8 * 512 * 512
_NC = 2
_NS = 16
_NW = _NC * _NS
_N_SUB = _N_TOTAL // _NW
_CHUNK = 16384
_NCHUNK = _N_SUB // _CHUNK
_COLS = 512
_ROWS = _N_TOTAL // _COLS
_CHUNK_ROWS = _CHUNK // _COLS
_ROWS_SUB = _N_SUB // _COLS
_UNROLL = 4
_LANES = 16

_mesh = plsc.VectorSubcoreMesh(core_axis_name="c", subcore_axis_name="s")


def _neg_log_clip(x):
    bits = plsc.bitcast(x, jnp.int32)
    ne = 127 - (bits >> 23)
    m = plsc.bitcast(bits | 0x3F800000, jnp.float32)
    s = (1.0 - m) / (m + 1.0)
    s2 = s * s
    poly = 1.0 + s2 * ((1.0 / 3.0) + s2 * ((1.0 / 5.0) + s2 * (1.0 / 7.0)))
    nlg = (2.0 * s) * poly + ne.astype(jnp.float32) * _LN2
    nlg = jnp.where(x > 0.0, nlg, 100.0)
    return jnp.minimum(nlg, 100.0)


def _loss_parts(pbuf, gbuf, mbuf, slot, r, c):
    p = pbuf[slot, r, pl.ds(c, _LANES)]
    g = gbuf[slot, r, pl.ds(c, _LANES)]
    m = mbuf[slot, r, pl.ds(c, _LANES)]
    pos = g * m
    x = jnp.where(g > 0.5, p, 1.0 - p)
    loss = _neg_log_clip(x)
    return pos, m, loss


def _stream_loop(pred_hbm, gt_hbm, mask_hbm, bufs, sems, base, init, accum_fn):
    pbuf, gbuf, mbuf = bufs

    def fire(ci, slot):
        r0 = base + ci * _CHUNK_ROWS
        sem = sems[slot]
        return (
            pltpu.async_copy(pred_hbm.at[pl.ds(r0, _CHUNK_ROWS)], pbuf.at[slot], sem),
            pltpu.async_copy(gt_hbm.at[pl.ds(r0, _CHUNK_ROWS)], gbuf.at[slot], sem),
            pltpu.async_copy(mask_hbm.at[pl.ds(r0, _CHUNK_ROWS)], mbuf.at[slot], sem),
        )

    carry = tuple(init for _ in range(_UNROLL))
    descs = fire(0, 0)
    for ci in range(_NCHUNK):
        slot = ci % 2
        for d in descs:
            d.wait()
        if ci + 1 < _NCHUNK:
            descs = fire(ci + 1, (ci + 1) % 2)

        def body(i, acc, _slot=slot):
            out = []
            for u in range(_UNROLL):
                off = i + u * _LANES
                r = off >> 9
                c = off & (_COLS - 1)
                pos, m, loss = _loss_parts(pbuf, gbuf, mbuf, _slot, r, c)
                out.append(accum_fn(acc[u], pos, m, loss))
            return tuple(out)

        carry = plsc.parallel_loop(
            0, _CHUNK, step=_UNROLL * _LANES, carry=carry)(body)

    def _merge(*accs):
        tot = accs[0]
        for a in accs[1:]:
            tot = jax.tree.map(lambda x, y: x + y, tot, a)
        return tot

    return _merge(*carry)


@functools.partial(
    pl.kernel,
    mesh=_mesh,
    compiler_params=pltpu.CompilerParams(needs_layout_passes=False),
    out_type=jax.ShapeDtypeStruct((_NW, 4 * _LANES), jnp.float32),
    scratch_types=[
        pltpu.VMEM((2, _CHUNK_ROWS, _COLS), jnp.float32),
        pltpu.VMEM((2, _CHUNK_ROWS, _COLS), jnp.float32),
        pltpu.VMEM((2, _CHUNK_ROWS, _COLS), jnp.float32),
        pltpu.VMEM((4 * _LANES,), jnp.float32),
        pltpu.SemaphoreType.DMA,
        pltpu.SemaphoreType.DMA,
    ],
)
def _sums_kernel(pred_hbm, gt_hbm, mask_hbm, out_hbm,
                 pbuf, gbuf, mbuf, obuf, sem_a, sem_b):
    wid = lax.axis_index("s") * _NC + lax.axis_index("c")
    base = wid * _ROWS_SUB
    z = jnp.zeros((_LANES,), jnp.float32)

    def accum(acc, pos, m, loss):
        pc, mc, pls, lsm = acc
        return (pc + pos, mc + m, pls + loss * pos, lsm + loss * m)

    pc, mc, pls, lsm = _stream_loop(
        pred_hbm, gt_hbm, mask_hbm, (pbuf, gbuf, mbuf), (sem_a, sem_b),
        base, (z, z, z, z), accum)

    obuf[pl.ds(0, _LANES)] = pc
    obuf[pl.ds(_LANES, _LANES)] = mc
    obuf[pl.ds(2 * _LANES, _LANES)] = pls
    obuf[pl.ds(3 * _LANES, _LANES)] = lsm
    pltpu.sync_copy(obuf, out_hbm.at[wid])


@functools.partial(
    pl.kernel,
    mesh=_mesh,
    compiler_params=pltpu.CompilerParams(needs_layout_passes=False),
    out_type=jax.ShapeDtypeStruct((_NW, 2 * _LANES), jnp.float32),
    scratch_types=[
        pltpu.VMEM((2, _CHUNK_ROWS, _COLS), jnp.float32),
        pltpu.VMEM((2, _CHUNK_ROWS, _COLS), jnp.float32),
        pltpu.VMEM((2, _CHUNK_ROWS, _COLS), jnp.float32),
        pltpu.VMEM((2 * _LANES,), jnp.float32),
        pltpu.VMEM((_LANES,), jnp.float32),
        pltpu.SemaphoreType.DMA,
        pltpu.SemaphoreType.DMA,
    ],
)
def _thresh_kernel(pred_hbm, gt_hbm, mask_hbm, thr_hbm, out_hbm,
                   pbuf, gbuf, mbuf, obuf, tbuf, sem_a, sem_b):
    wid = lax.axis_index("s") * _NC + lax.axis_index("c")
    base = wid * _ROWS_SUB
    pltpu.sync_copy(thr_hbm, tbuf)
    thr = tbuf[...]
    z = jnp.zeros((_LANES,), jnp.float32)

    def accum(acc, pos, m, loss):
        cnt, ssum = acc
        v = loss * (m - pos)
        sel = v > thr
        return (cnt + jnp.where(sel, 1.0, 0.0), ssum + jnp.where(sel, v, 0.0))

    cnt, ssum = _stream_loop(
        pred_hbm, gt_hbm, mask_hbm, (pbuf, gbuf, mbuf), (sem_a, sem_b),
        base, (z, z), accum)

    obuf[pl.ds(0, _LANES)] = cnt
    obuf[pl.ds(_LANES, _LANES)] = ssum
    pltpu.sync_copy(obuf, out_hbm.at[wid])


def kernel(pred, gt, mask):
    p = pred.reshape(_ROWS, _COLS)
    g = gt.reshape(_ROWS, _COLS)
    m = mask.reshape(_ROWS, _COLS)

    parts = _sums_kernel(p, g, m).reshape(_NW, 4, _LANES)
    pos_cnt = parts[:, 0].sum()
    neg_cnt = parts[:, 1].sum() - pos_cnt
    pos_loss_sum = parts[:, 2].sum()
    neg_loss_sum = parts[:, 3].sum() - pos_loss_sum

    pos_count = jnp.floor(pos_cnt)
    neg_floor = jnp.floor(neg_cnt)
    negative_count = jnp.minimum(neg_floor, jnp.floor(pos_count * _NEGATIVE_RATIO))

    def scan_at(bits_i32):
        t = lax.bitcast_convert_type(bits_i32, jnp.float32)
        sp = _thresh_kernel(p, g, m, jnp.broadcast_to(t, (_LANES,)))
        sp = sp.reshape(_NW, 2, _LANES)
        return sp[:, 0].sum(), sp[:, 1].sum()

    def topk_rare():
        def bs(_, lohi):
            lo, hi = lohi
            mid = (lo + hi) // 2
            cnt, _s = scan_at(mid)
            take = cnt < negative_count
            return (jnp.where(take, lo, mid + 1), jnp.where(take, mid, hi))

        lo, _ = lax.fori_loop(0, 31, bs, (jnp.int32(0), jnp.int32(0x7F800000)))
        vk = lax.bitcast_convert_type(lo, jnp.float32)
        cnt, ssum = scan_at(lo)
        val = ssum + (negative_count - cnt) * vk
        return jnp.where(negative_count > 0.0, val, 0.0)

    def topk_fast():
        return neg_loss_sum

    neg_top_sum = lax.cond(negative_count < neg_floor, topk_rare, topk_fast)
    return (pos_loss_sum + neg_top_sum) / (pos_count + negative_count + _EPS)
